# baseline (speedup 1.0000x reference)
import jax
import jax.numpy as jnp
from jax.experimental import pallas as pl

_EPS = 1e-8


def _simrel_kernel(x_ref, ca_ref, out_ref):
    x = x_ref[...]
    ca = ca_ref[...]
    inv_in = 1.0 / jnp.maximum(jnp.sqrt(jnp.sum(x * x, axis=1, keepdims=True)), _EPS)
    inv_ca = 1.0 / jnp.maximum(jnp.sqrt(jnp.sum(ca * ca, axis=1)), _EPS)
    dots = jax.lax.dot_general(
        x, ca,
        dimension_numbers=(((1,), (1,)), ((), ())),
        preferred_element_type=jnp.float32,
    )
    out_ref[...] = dots * inv_in * inv_ca[None, :]


def kernel(inputs, labels, class_avgs):
    del labels
    b, s, d = inputs.shape
    k = class_avgs.shape[0]
    x = inputs.reshape(b * s, d)
    out = pl.pallas_call(
        _simrel_kernel,
        out_shape=jax.ShapeDtypeStruct((b * s, k), jnp.float32),
    )(x, class_avgs)
    return out.reshape(b, s, k)
